# Initial kernel scaffold; baseline (speedup 1.0000x reference)
import jax
import jax.numpy as jnp
from jax.experimental import pallas as pl


def kernel(x, edge_index, W1, b1, W2, b2, W3, b3, W4, b4):
    raise NotImplementedError("write your pallas kernel here")


# speedup vs baseline: 4.3657x; 4.3657x over previous
import functools

import jax
import jax.numpy as jnp
from jax import lax
from jax.experimental import pallas as pl
from jax.experimental.pallas import tpu as pltpu
from jax.experimental.pallas import tpu_sc as plsc

_NC = 2
_NS = 16
_CH = 80
_DW = 16


def _sc_mesh():
    return plsc.VectorSubcoreMesh(core_axis_name="c", subcore_axis_name="s")


def _make_agg(N, E, D):
    e_core = E // _NC
    e_tile = e_core // _NS
    nch = e_tile // _CH
    rows_tile = (N // _NS) // 8 * 8
    rem = N - _NS * rows_tile

    @functools.partial(
        pl.kernel,
        mesh=_sc_mesh(),
        out_type=jax.ShapeDtypeStruct((_NC * N, D), jnp.float32),
        scratch_types=[
            pltpu.VMEM((_CH,), jnp.int32),
            pltpu.VMEM((_CH,), jnp.int32),
            pltpu.VMEM((_CH, D), jnp.float32),
            pltpu.VMEM_SHARED((N, D), jnp.float32),
            pltpu.SemaphoreType.DMA,
        ],
    )
    def agg(h_hbm, src_hbm, dst_hbm, zero_hbm, out_hbm,
            src_v, dst_v, rows_v, acc_sh, sem):
        c = lax.axis_index("c")
        s = lax.axis_index("s")
        r0 = s * rows_tile
        pltpu.sync_copy(zero_hbm.at[pl.ds(r0, rows_tile)],
                        acc_sh.at[pl.ds(r0, rows_tile)])
        if rem:
            @pl.when(s == _NS - 1)
            def _zero_rem():
                pltpu.sync_copy(zero_hbm.at[pl.ds(N - rem, rem)],
                                acc_sh.at[pl.ds(N - rem, rem)])
        plsc.subcore_barrier()
        base = c * e_core + s * e_tile

        def body(j, carry):
            off = base + j * _CH
            pltpu.sync_copy(src_hbm.at[pl.ds(off, _CH)], src_v)
            pltpu.sync_copy(dst_hbm.at[pl.ds(off, _CH)], dst_v)
            pltpu.async_copy(h_hbm.at[src_v], rows_v, sem).wait()
            pltpu.sync_copy(rows_v, acc_sh.at[dst_v], add=True)
            return carry

        lax.fori_loop(0, nch, body, 0)
        plsc.subcore_barrier()
        pltpu.sync_copy(acc_sh.at[pl.ds(r0, rows_tile)],
                        out_hbm.at[pl.ds(c * N + r0, rows_tile)])
        if rem:
            @pl.when(s == _NS - 1)
            def _out_rem():
                pltpu.sync_copy(acc_sh.at[pl.ds(N - rem, rem)],
                                out_hbm.at[pl.ds(c * N + N - rem, rem)])

    return agg


def _make_cnt(N, E, D):
    e_core = E // _NC
    e_tile = e_core // _NS
    nch = e_tile // _CH
    rows_tile = (N // _NS) // 8 * 8
    rem = N - _NS * rows_tile

    @functools.partial(
        pl.kernel,
        mesh=_sc_mesh(),
        out_type=jax.ShapeDtypeStruct((_NC * N, D), jnp.float32),
        scratch_types=[
            pltpu.VMEM((_CH,), jnp.int32),
            pltpu.VMEM((_CH, D), jnp.float32),
            pltpu.VMEM_SHARED((N, D), jnp.float32),
        ],
    )
    def cnt(idx_hbm, ones_hbm, zero_hbm, out_hbm, idx_v, rows_v, acc_sh):
        c = lax.axis_index("c")
        s = lax.axis_index("s")
        r0 = s * rows_tile
        pltpu.sync_copy(zero_hbm.at[pl.ds(r0, rows_tile)],
                        acc_sh.at[pl.ds(r0, rows_tile)])
        if rem:
            @pl.when(s == _NS - 1)
            def _zero_rem():
                pltpu.sync_copy(zero_hbm.at[pl.ds(N - rem, rem)],
                                acc_sh.at[pl.ds(N - rem, rem)])
        pltpu.sync_copy(ones_hbm, rows_v)
        plsc.subcore_barrier()
        base = c * e_core + s * e_tile

        def body(j, carry):
            off = base + j * _CH
            pltpu.sync_copy(idx_hbm.at[pl.ds(off, _CH)], idx_v)
            pltpu.sync_copy(rows_v, acc_sh.at[idx_v], add=True)
            return carry

        lax.fori_loop(0, nch, body, 0)
        plsc.subcore_barrier()
        pltpu.sync_copy(acc_sh.at[pl.ds(r0, rows_tile)],
                        out_hbm.at[pl.ds(c * N + r0, rows_tile)])
        if rem:
            @pl.when(s == _NS - 1)
            def _out_rem():
                pltpu.sync_copy(acc_sh.at[pl.ds(N - rem, rem)],
                                out_hbm.at[pl.ds(c * N + N - rem, rem)])

    return cnt


def _prep(x, dop, dip, N, D):
    BN = 400
    G = N // BN

    def body(x_ref, do_ref, di_ref, no_ref, ni_ref, hs_ref):
        do = do_ref[0, :, 0:1] + do_ref[1, :, 0:1]
        di = di_ref[0, :, 0:1] + di_ref[1, :, 0:1]
        no = lax.rsqrt(jnp.maximum(do, 1.0))
        ni = lax.rsqrt(jnp.maximum(di, 1.0))
        no_ref[...] = no
        ni_ref[...] = ni
        hs_ref[...] = x_ref[...] * no

    return pl.pallas_call(
        body,
        grid=(G,),
        in_specs=[
            pl.BlockSpec((BN, D), lambda i: (i, 0)),
            pl.BlockSpec((2, BN, D), lambda i: (0, i, 0)),
            pl.BlockSpec((2, BN, D), lambda i: (0, i, 0)),
        ],
        out_specs=[
            pl.BlockSpec((BN, 1), lambda i: (i, 0)),
            pl.BlockSpec((BN, 1), lambda i: (i, 0)),
            pl.BlockSpec((BN, D), lambda i: (i, 0)),
        ],
        out_shape=[
            jax.ShapeDtypeStruct((N, 1), jnp.float32),
            jax.ShapeDtypeStruct((N, 1), jnp.float32),
            jax.ShapeDtypeStruct((N, D), jnp.float32),
        ],
    )(x, dop, dip)


def _layer(aggp, ni, no, W, b, N, D):
    BN = 400
    G = N // BN

    def body(a_ref, ni_ref, no_ref, w_ref, b_ref, o_ref):
        a = (a_ref[0] + a_ref[1]) * ni_ref[...]
        h = jnp.dot(a, w_ref[...], preferred_element_type=jnp.float32)
        h = h + b_ref[...]
        o_ref[...] = jnp.maximum(h, 0.0) * no_ref[...]

    return pl.pallas_call(
        body,
        grid=(G,),
        in_specs=[
            pl.BlockSpec((2, BN, D), lambda i: (0, i, 0)),
            pl.BlockSpec((BN, 1), lambda i: (i, 0)),
            pl.BlockSpec((BN, 1), lambda i: (i, 0)),
            pl.BlockSpec((D, D), lambda i: (0, 0)),
            pl.BlockSpec((1, D), lambda i: (0, 0)),
        ],
        out_specs=pl.BlockSpec((BN, D), lambda i: (i, 0)),
        out_shape=jax.ShapeDtypeStruct((N, D), jnp.float32),
    )(aggp, ni, no, W, b.reshape(1, D))


def _final(aggp, ni, W, b, N, D):
    BN = 400
    G = N // BN

    def body(a_ref, ni_ref, w_ref, b_ref, o_ref, acc_ref):
        i = pl.program_id(0)
        a = (a_ref[0] + a_ref[1]) * ni_ref[...]
        p = jnp.sum(a, axis=0, keepdims=True)

        @pl.when(i == 0)
        def _():
            acc_ref[...] = p

        @pl.when(i > 0)
        def _():
            acc_ref[...] = acc_ref[...] + p

        @pl.when(i == G - 1)
        def _():
            v = acc_ref[...] * (1.0 / N)
            o_ref[...] = jnp.dot(v, w_ref[...],
                                 preferred_element_type=jnp.float32) + b_ref[...]

    return pl.pallas_call(
        body,
        grid=(G,),
        in_specs=[
            pl.BlockSpec((2, BN, D), lambda i: (0, i, 0)),
            pl.BlockSpec((BN, 1), lambda i: (i, 0)),
            pl.BlockSpec((D, D), lambda i: (0, 0)),
            pl.BlockSpec((1, D), lambda i: (0, 0)),
        ],
        out_specs=pl.BlockSpec((1, D), lambda i: (0, 0)),
        out_shape=jax.ShapeDtypeStruct((1, D), jnp.float32),
        scratch_shapes=[pltpu.VMEM((1, D), jnp.float32)],
    )(aggp, ni, W, b.reshape(1, D))


def kernel(x, edge_index, W1, b1, W2, b2, W3, b3, W4, b4):
    N, D = x.shape
    E = edge_index.shape[1]
    src = edge_index[0].astype(jnp.int32)
    dst = edge_index[1].astype(jnp.int32)

    zero_nd = jnp.zeros((N, D), jnp.float32)
    ones_ch = jnp.ones((_CH, D), jnp.float32)

    cnt_fn = _make_cnt(N, E, D)
    dop = cnt_fn(src, ones_ch, zero_nd).reshape(_NC, N, D)
    dip = cnt_fn(dst, ones_ch, zero_nd).reshape(_NC, N, D)
    no, ni, hs = _prep(x, dop, dip, N, D)

    agg_fn = _make_agg(N, E, D)
    for W, b in ((W1, b1), (W2, b2), (W3, b3)):
        aggp = agg_fn(hs, src, dst, zero_nd).reshape(_NC, N, D)
        hs = _layer(aggp, ni, no, W, b, N, D)
    aggp = agg_fn(hs, src, dst, zero_nd).reshape(_NC, N, D)
    return _final(aggp, ni, W4, b4, N, D)


# speedup vs baseline: 9.1247x; 2.0901x over previous
import functools

import jax
import jax.numpy as jnp
from jax import lax
from jax.experimental import pallas as pl
from jax.experimental.pallas import tpu as pltpu
from jax.experimental.pallas import tpu_sc as plsc

_NC = 2
_NS = 16
_CH = 128


def _sc_mesh():
    return plsc.VectorSubcoreMesh(core_axis_name="c", subcore_axis_name="s")


def _make_agg(N, E, D):
    e_core = E // _NC
    e_tile = e_core // _NS
    nfull = e_tile // _CH
    tail = e_tile - nfull * _CH
    rows_tile = (N // _NS) // 8 * 8
    rem = N - _NS * rows_tile
    assert nfull >= 3 and nfull % 2 == 0 and tail % 8 == 0

    @functools.partial(
        pl.kernel,
        mesh=_sc_mesh(),
        out_type=jax.ShapeDtypeStruct((_NC * N, D), jnp.float32),
        scratch_types=[
            pltpu.VMEM((_CH,), jnp.int32),
            pltpu.VMEM((_CH,), jnp.int32),
            pltpu.VMEM((_CH,), jnp.int32),
            pltpu.VMEM((_CH,), jnp.int32),
            pltpu.VMEM((_CH, D), jnp.float32),
            pltpu.VMEM((_CH, D), jnp.float32),
            pltpu.VMEM((tail,), jnp.int32),
            pltpu.VMEM((tail,), jnp.int32),
            pltpu.VMEM((tail, D), jnp.float32),
            pltpu.VMEM_SHARED((N, D), jnp.float32),
            pltpu.SemaphoreType.DMA,
            pltpu.SemaphoreType.DMA,
            pltpu.SemaphoreType.DMA,
            pltpu.SemaphoreType.DMA,
        ],
    )
    def agg(h_hbm, src_hbm, dst_hbm, zero_hbm, out_hbm,
            s0, s1, d0, d1, r0b, r1b, st, dt, rt, acc_sh,
            si0, si1, sg0, sg1):
        c = lax.axis_index("c")
        s = lax.axis_index("s")
        r0 = s * rows_tile
        pltpu.sync_copy(zero_hbm.at[pl.ds(r0, rows_tile)],
                        acc_sh.at[pl.ds(r0, rows_tile)])
        if rem:
            @pl.when(s == _NS - 1)
            def _zero_rem():
                pltpu.sync_copy(zero_hbm.at[pl.ds(N - rem, rem)],
                                acc_sh.at[pl.ds(N - rem, rem)])
        plsc.subcore_barrier()
        base = c * e_core + s * e_tile

        def fire_idx(off, s_b, d_b, sem):
            pltpu.async_copy(src_hbm.at[pl.ds(off, _CH)], s_b, sem)
            pltpu.async_copy(dst_hbm.at[pl.ds(off, _CH)], d_b, sem)

        def wait_idx(s_b, d_b, sem):
            pltpu.make_async_copy(src_hbm.at[pl.ds(0, _CH)], s_b, sem).wait()
            pltpu.make_async_copy(src_hbm.at[pl.ds(0, _CH)], d_b, sem).wait()

        def wait_rows(r_b, sem):
            pltpu.make_async_copy(h_hbm.at[pl.ds(0, _CH)], r_b, sem).wait()

        def half(next_off, cur, oth, first=False, prefetch=True):
            s_c, d_c, r_c, si_c, sg_c = cur
            s_o, d_o, r_o, si_o, sg_o = oth
            wait_idx(s_c, d_c, si_c)
            pltpu.async_copy(h_hbm.at[s_c], r_c, sg_c)
            if not first:
                wait_rows(r_o, sg_o)
                pltpu.sync_copy(r_o, acc_sh.at[d_o], add=True)
            if prefetch:
                fire_idx(next_off, s_o, d_o, si_o)

        p0 = (s0, d0, r0b, si0, sg0)
        p1 = (s1, d1, r1b, si1, sg1)

        fire_idx(base, s0, d0, si0)
        half(base + _CH, p0, p1, first=True)

        def body(j, carry):
            off1 = base + (2 * j + 1) * _CH
            half(off1 + _CH, p1, p0)
            half(off1 + 2 * _CH, p0, p1)
            return carry

        lax.fori_loop(0, (nfull - 2) // 2, body, 0)

        half(0, p1, p0, prefetch=False)
        wait_rows(r1b, sg1)
        pltpu.sync_copy(r1b, acc_sh.at[d1], add=True)

        if tail:
            toff = base + nfull * _CH
            pltpu.sync_copy(src_hbm.at[pl.ds(toff, tail)], st)
            pltpu.sync_copy(dst_hbm.at[pl.ds(toff, tail)], dt)
            pltpu.async_copy(h_hbm.at[st], rt, sg0)
            pltpu.make_async_copy(h_hbm.at[pl.ds(0, tail)], rt, sg0).wait()
            pltpu.sync_copy(rt, acc_sh.at[dt], add=True)

        plsc.subcore_barrier()
        pltpu.sync_copy(acc_sh.at[pl.ds(r0, rows_tile)],
                        out_hbm.at[pl.ds(c * N + r0, rows_tile)])
        if rem:
            @pl.when(s == _NS - 1)
            def _out_rem():
                pltpu.sync_copy(acc_sh.at[pl.ds(N - rem, rem)],
                                out_hbm.at[pl.ds(c * N + N - rem, rem)])

    return agg


def _make_cnt(N, E, D):
    e_core = E // _NC
    e_tile = e_core // _NS
    nfull = e_tile // _CH
    tail = e_tile - nfull * _CH
    rows_tile = (N // _NS) // 8 * 8
    rem = N - _NS * rows_tile
    assert nfull >= 3 and nfull % 2 == 0 and tail % 8 == 0

    @functools.partial(
        pl.kernel,
        mesh=_sc_mesh(),
        out_type=jax.ShapeDtypeStruct((_NC * N, D), jnp.float32),
        scratch_types=[
            pltpu.VMEM((_CH,), jnp.int32),
            pltpu.VMEM((_CH,), jnp.int32),
            pltpu.VMEM((tail,), jnp.int32),
            pltpu.VMEM((_CH, D), jnp.float32),
            pltpu.VMEM((tail, D), jnp.float32),
            pltpu.VMEM_SHARED((N, D), jnp.float32),
            pltpu.SemaphoreType.DMA,
            pltpu.SemaphoreType.DMA,
        ],
    )
    def cnt(idx_hbm, ones_hbm, zero_hbm, out_hbm,
            d0, d1, dt, rows_v, rows_t, acc_sh, si0, si1):
        c = lax.axis_index("c")
        s = lax.axis_index("s")
        r0 = s * rows_tile
        pltpu.sync_copy(zero_hbm.at[pl.ds(r0, rows_tile)],
                        acc_sh.at[pl.ds(r0, rows_tile)])
        if rem:
            @pl.when(s == _NS - 1)
            def _zero_rem():
                pltpu.sync_copy(zero_hbm.at[pl.ds(N - rem, rem)],
                                acc_sh.at[pl.ds(N - rem, rem)])
        pltpu.sync_copy(ones_hbm, rows_v)
        if tail:
            pltpu.sync_copy(ones_hbm.at[pl.ds(0, tail)], rows_t)
        plsc.subcore_barrier()
        base = c * e_core + s * e_tile

        def wait_idx(d_b, sem):
            pltpu.make_async_copy(idx_hbm.at[pl.ds(0, _CH)], d_b, sem).wait()

        def half(next_off, d_c, si_c, d_o, si_o, prefetch=True):
            wait_idx(d_c, si_c)
            if prefetch:
                pltpu.async_copy(idx_hbm.at[pl.ds(next_off, _CH)], d_o, si_o)
            pltpu.sync_copy(rows_v, acc_sh.at[d_c], add=True)

        pltpu.async_copy(idx_hbm.at[pl.ds(base, _CH)], d0, si0)
        half(base + _CH, d0, si0, d1, si1)

        def body(j, carry):
            off1 = base + (2 * j + 1) * _CH
            half(off1 + _CH, d1, si1, d0, si0)
            half(off1 + 2 * _CH, d0, si0, d1, si1)
            return carry

        lax.fori_loop(0, (nfull - 2) // 2, body, 0)

        half(0, d1, si1, d0, si0, prefetch=False)
        if tail:
            toff = base + nfull * _CH
            pltpu.sync_copy(idx_hbm.at[pl.ds(toff, tail)], dt)
            pltpu.sync_copy(rows_t, acc_sh.at[dt], add=True)
        plsc.subcore_barrier()
        pltpu.sync_copy(acc_sh.at[pl.ds(r0, rows_tile)],
                        out_hbm.at[pl.ds(c * N + r0, rows_tile)])
        if rem:
            @pl.when(s == _NS - 1)
            def _out_rem():
                pltpu.sync_copy(acc_sh.at[pl.ds(N - rem, rem)],
                                out_hbm.at[pl.ds(c * N + N - rem, rem)])

    return cnt


def _prep(x, dop, dip, N, D):
    BN = 400
    G = N // BN

    def body(x_ref, do_ref, di_ref, no_ref, ni_ref, hs_ref):
        do = do_ref[0, :, 0:1] + do_ref[1, :, 0:1]
        di = di_ref[0, :, 0:1] + di_ref[1, :, 0:1]
        no = lax.rsqrt(jnp.maximum(do, 1.0))
        ni = lax.rsqrt(jnp.maximum(di, 1.0))
        no_ref[...] = no
        ni_ref[...] = ni
        hs_ref[...] = x_ref[...] * no

    return pl.pallas_call(
        body,
        grid=(G,),
        in_specs=[
            pl.BlockSpec((BN, D), lambda i: (i, 0)),
            pl.BlockSpec((2, BN, D), lambda i: (0, i, 0)),
            pl.BlockSpec((2, BN, D), lambda i: (0, i, 0)),
        ],
        out_specs=[
            pl.BlockSpec((BN, 1), lambda i: (i, 0)),
            pl.BlockSpec((BN, 1), lambda i: (i, 0)),
            pl.BlockSpec((BN, D), lambda i: (i, 0)),
        ],
        out_shape=[
            jax.ShapeDtypeStruct((N, 1), jnp.float32),
            jax.ShapeDtypeStruct((N, 1), jnp.float32),
            jax.ShapeDtypeStruct((N, D), jnp.float32),
        ],
    )(x, dop, dip)


def _layer(aggp, ni, no, W, b, N, D):
    BN = 400
    G = N // BN

    def body(a_ref, ni_ref, no_ref, w_ref, b_ref, o_ref):
        a = (a_ref[0] + a_ref[1]) * ni_ref[...]
        h = jnp.dot(a, w_ref[...], preferred_element_type=jnp.float32)
        h = h + b_ref[...]
        o_ref[...] = jnp.maximum(h, 0.0) * no_ref[...]

    return pl.pallas_call(
        body,
        grid=(G,),
        in_specs=[
            pl.BlockSpec((2, BN, D), lambda i: (0, i, 0)),
            pl.BlockSpec((BN, 1), lambda i: (i, 0)),
            pl.BlockSpec((BN, 1), lambda i: (i, 0)),
            pl.BlockSpec((D, D), lambda i: (0, 0)),
            pl.BlockSpec((1, D), lambda i: (0, 0)),
        ],
        out_specs=pl.BlockSpec((BN, D), lambda i: (i, 0)),
        out_shape=jax.ShapeDtypeStruct((N, D), jnp.float32),
    )(aggp, ni, no, W, b.reshape(1, D))


def _final(aggp, ni, W, b, N, D):
    BN = 400
    G = N // BN

    def body(a_ref, ni_ref, w_ref, b_ref, o_ref, acc_ref):
        i = pl.program_id(0)
        a = (a_ref[0] + a_ref[1]) * ni_ref[...]
        p = jnp.sum(a, axis=0, keepdims=True)

        @pl.when(i == 0)
        def _():
            acc_ref[...] = p

        @pl.when(i > 0)
        def _():
            acc_ref[...] = acc_ref[...] + p

        @pl.when(i == G - 1)
        def _():
            v = acc_ref[...] * (1.0 / N)
            o_ref[...] = jnp.dot(v, w_ref[...],
                                 preferred_element_type=jnp.float32) + b_ref[...]

    return pl.pallas_call(
        body,
        grid=(G,),
        in_specs=[
            pl.BlockSpec((2, BN, D), lambda i: (0, i, 0)),
            pl.BlockSpec((BN, 1), lambda i: (i, 0)),
            pl.BlockSpec((D, D), lambda i: (0, 0)),
            pl.BlockSpec((1, D), lambda i: (0, 0)),
        ],
        out_specs=pl.BlockSpec((1, D), lambda i: (0, 0)),
        out_shape=jax.ShapeDtypeStruct((1, D), jnp.float32),
        scratch_shapes=[pltpu.VMEM((1, D), jnp.float32)],
    )(aggp, ni, W, b.reshape(1, D))


def kernel(x, edge_index, W1, b1, W2, b2, W3, b3, W4, b4):
    N, D = x.shape
    E = edge_index.shape[1]
    src = edge_index[0].astype(jnp.int32)
    dst = edge_index[1].astype(jnp.int32)

    zero_nd = jnp.zeros((N, D), jnp.float32)
    ones_ch = jnp.ones((_CH, D), jnp.float32)

    cnt_fn = _make_cnt(N, E, D)
    dop = cnt_fn(src, ones_ch, zero_nd).reshape(_NC, N, D)
    dip = cnt_fn(dst, ones_ch, zero_nd).reshape(_NC, N, D)
    no, ni, hs = _prep(x, dop, dip, N, D)

    agg_fn = _make_agg(N, E, D)
    for W, b in ((W1, b1), (W2, b2), (W3, b3)):
        aggp = agg_fn(hs, src, dst, zero_nd).reshape(_NC, N, D)
        hs = _layer(aggp, ni, no, W, b, N, D)
    aggp = agg_fn(hs, src, dst, zero_nd).reshape(_NC, N, D)
    return _final(aggp, ni, W4, b4, N, D)


# speedup vs baseline: 9.8819x; 1.0830x over previous
import functools

import jax
import jax.numpy as jnp
from jax import lax
from jax.experimental import pallas as pl
from jax.experimental.pallas import tpu as pltpu
from jax.experimental.pallas import tpu_sc as plsc

_NC = 2
_NS = 16
_CH = 128


def _sc_mesh():
    return plsc.VectorSubcoreMesh(core_axis_name="c", subcore_axis_name="s")


def _make_agg(N, E, D):
    CHA = 64
    e_core = E // _NC
    e_tile = e_core // _NS
    nfull = e_tile // CHA
    tail = e_tile - nfull * CHA
    rows_tile = (N // _NS) // 8 * 8
    rem = N - _NS * rows_tile
    epi = (nfull - 6) % 4 + 3
    nloop = (nfull - 3 - epi) // 4
    assert nfull >= 3 + epi and tail % 8 == 0

    @functools.partial(
        pl.kernel,
        mesh=_sc_mesh(),
        out_type=jax.ShapeDtypeStruct((_NC * N, D), jnp.float32),
        scratch_types=(
            [pltpu.VMEM((CHA,), jnp.int32)] * 4 +
            [pltpu.VMEM((CHA,), jnp.int32)] * 4 +
            [pltpu.VMEM((CHA, D), jnp.float32)] * 4 +
            [pltpu.VMEM((tail,), jnp.int32)] * 2 +
            [pltpu.VMEM((tail, D), jnp.float32)] +
            [pltpu.VMEM_SHARED((N, D), jnp.float32)] +
            [pltpu.SemaphoreType.DMA] * 12
        ),
    )
    def agg(h_hbm, src_hbm, dst_hbm, zero_hbm, out_hbm,
            sA, sB, sC, sD, dA, dB, dC, dD, rA, rB, rC, rD,
            st, dt, rt, acc_sh,
            siA, siB, siC, siD, sgA, sgB, sgC, sgD, ssA, ssB, ssC, ssD):
        c = lax.axis_index("c")
        s = lax.axis_index("s")
        r0 = s * rows_tile
        pltpu.sync_copy(zero_hbm.at[pl.ds(r0, rows_tile)],
                        acc_sh.at[pl.ds(r0, rows_tile)])
        if rem:
            @pl.when(s == _NS - 1)
            def _zero_rem():
                pltpu.sync_copy(zero_hbm.at[pl.ds(N - rem, rem)],
                                acc_sh.at[pl.ds(N - rem, rem)])
        plsc.subcore_barrier()
        base = c * e_core + s * e_tile

        sets = [(sA, dA, rA, siA, sgA, ssA), (sB, dB, rB, siB, sgB, ssB),
                (sC, dC, rC, siC, sgC, ssC), (sD, dD, rD, siD, sgD, ssD)]

        def fire_idx(off, t):
            pltpu.async_copy(src_hbm.at[pl.ds(off, CHA)], t[0], t[3])
            pltpu.async_copy(dst_hbm.at[pl.ds(off, CHA)], t[1], t[3])

        def wait_idx(t):
            pltpu.make_async_copy(src_hbm.at[pl.ds(0, CHA)], t[0], t[3]).wait()
            pltpu.make_async_copy(src_hbm.at[pl.ds(0, CHA)], t[1], t[3]).wait()

        def fire_gather(t):
            pltpu.async_copy(h_hbm.at[t[0]], t[2], t[4])

        def wait_gather(t):
            pltpu.make_async_copy(h_hbm.at[pl.ds(0, CHA)], t[2], t[4]).wait()

        def fire_scatter(t):
            pltpu.async_copy(t[2], acc_sh.at[t[1]], t[5], add=True)

        def wait_scatter(t):
            pltpu.make_async_copy(h_hbm.at[pl.ds(0, CHA)], t[2], t[5]).wait()

        fire_idx(base, sets[0])
        fire_idx(base + CHA, sets[1])
        wait_idx(sets[0])
        fire_gather(sets[0])
        fire_idx(base + 2 * CHA, sets[2])
        wait_idx(sets[1])
        fire_gather(sets[1])
        wait_gather(sets[0])
        fire_scatter(sets[0])
        fire_idx(base + 3 * CHA, sets[3])
        wait_idx(sets[2])
        fire_gather(sets[2])
        wait_gather(sets[1])
        fire_scatter(sets[1])
        wait_scatter(sets[0])
        fire_idx(base + 4 * CHA, sets[0])

        def body(k, carry):
            c0 = 3 + 4 * k
            for m in range(4):
                ch = c0 + m
                t = sets[(3 + m) % 4]
                tg = sets[(2 + m) % 4]
                tw = sets[(1 + m) % 4]
                wait_idx(t)
                fire_gather(t)
                wait_gather(tg)
                fire_scatter(tg)
                wait_scatter(tw)
                fire_idx(base + (ch + 2) * CHA, tw)
            return carry

        lax.fori_loop(0, nloop, body, 0)

        cL = nfull - 1
        for ch in range(3 + 4 * nloop, nfull):
            t = sets[ch % 4]
            tg = sets[(ch - 1) % 4]
            tw = sets[(ch - 2) % 4]
            wait_idx(t)
            fire_gather(t)
            wait_gather(tg)
            fire_scatter(tg)
            wait_scatter(tw)
            if ch + 2 <= cL:
                fire_idx(base + (ch + 2) * CHA, tw)
        wait_gather(sets[cL % 4])
        fire_scatter(sets[cL % 4])
        wait_scatter(sets[(cL - 1) % 4])
        wait_scatter(sets[cL % 4])

        if tail:
            toff = base + nfull * CHA
            pltpu.sync_copy(src_hbm.at[pl.ds(toff, tail)], st)
            pltpu.sync_copy(dst_hbm.at[pl.ds(toff, tail)], dt)
            pltpu.async_copy(h_hbm.at[st], rt, sgA)
            pltpu.make_async_copy(h_hbm.at[pl.ds(0, tail)], rt, sgA).wait()
            pltpu.sync_copy(rt, acc_sh.at[dt], add=True)

        plsc.subcore_barrier()
        pltpu.sync_copy(acc_sh.at[pl.ds(r0, rows_tile)],
                        out_hbm.at[pl.ds(c * N + r0, rows_tile)])
        if rem:
            @pl.when(s == _NS - 1)
            def _out_rem():
                pltpu.sync_copy(acc_sh.at[pl.ds(N - rem, rem)],
                                out_hbm.at[pl.ds(c * N + N - rem, rem)])

    return agg


def _make_cnt(N, E, D):
    e_core = E // _NC
    e_tile = e_core // _NS
    nfull = e_tile // _CH
    tail = e_tile - nfull * _CH
    rows_tile = (N // _NS) // 8 * 8
    rem = N - _NS * rows_tile
    assert nfull >= 3 and nfull % 2 == 0 and tail % 8 == 0

    @functools.partial(
        pl.kernel,
        mesh=_sc_mesh(),
        out_type=jax.ShapeDtypeStruct((_NC * N, D), jnp.float32),
        scratch_types=[
            pltpu.VMEM((_CH,), jnp.int32),
            pltpu.VMEM((_CH,), jnp.int32),
            pltpu.VMEM((tail,), jnp.int32),
            pltpu.VMEM((_CH, D), jnp.float32),
            pltpu.VMEM((tail, D), jnp.float32),
            pltpu.VMEM_SHARED((N, D), jnp.float32),
            pltpu.SemaphoreType.DMA,
            pltpu.SemaphoreType.DMA,
        ],
    )
    def cnt(idx_hbm, ones_hbm, zero_hbm, out_hbm,
            d0, d1, dt, rows_v, rows_t, acc_sh, si0, si1):
        c = lax.axis_index("c")
        s = lax.axis_index("s")
        r0 = s * rows_tile
        pltpu.sync_copy(zero_hbm.at[pl.ds(r0, rows_tile)],
                        acc_sh.at[pl.ds(r0, rows_tile)])
        if rem:
            @pl.when(s == _NS - 1)
            def _zero_rem():
                pltpu.sync_copy(zero_hbm.at[pl.ds(N - rem, rem)],
                                acc_sh.at[pl.ds(N - rem, rem)])
        pltpu.sync_copy(ones_hbm, rows_v)
        if tail:
            pltpu.sync_copy(ones_hbm.at[pl.ds(0, tail)], rows_t)
        plsc.subcore_barrier()
        base = c * e_core + s * e_tile

        def wait_idx(d_b, sem):
            pltpu.make_async_copy(idx_hbm.at[pl.ds(0, _CH)], d_b, sem).wait()

        def half(next_off, d_c, si_c, d_o, si_o, prefetch=True):
            wait_idx(d_c, si_c)
            if prefetch:
                pltpu.async_copy(idx_hbm.at[pl.ds(next_off, _CH)], d_o, si_o)
            pltpu.sync_copy(rows_v, acc_sh.at[d_c], add=True)

        pltpu.async_copy(idx_hbm.at[pl.ds(base, _CH)], d0, si0)
        half(base + _CH, d0, si0, d1, si1)

        def body(j, carry):
            off1 = base + (2 * j + 1) * _CH
            half(off1 + _CH, d1, si1, d0, si0)
            half(off1 + 2 * _CH, d0, si0, d1, si1)
            return carry

        lax.fori_loop(0, (nfull - 2) // 2, body, 0)

        half(0, d1, si1, d0, si0, prefetch=False)
        if tail:
            toff = base + nfull * _CH
            pltpu.sync_copy(idx_hbm.at[pl.ds(toff, tail)], dt)
            pltpu.sync_copy(rows_t, acc_sh.at[dt], add=True)
        plsc.subcore_barrier()
        pltpu.sync_copy(acc_sh.at[pl.ds(r0, rows_tile)],
                        out_hbm.at[pl.ds(c * N + r0, rows_tile)])
        if rem:
            @pl.when(s == _NS - 1)
            def _out_rem():
                pltpu.sync_copy(acc_sh.at[pl.ds(N - rem, rem)],
                                out_hbm.at[pl.ds(c * N + N - rem, rem)])

    return cnt


def _prep(x, dop, dip, N, D):
    BN = 400
    G = N // BN

    def body(x_ref, do_ref, di_ref, no_ref, ni_ref, hs_ref):
        do = do_ref[0, :, 0:1] + do_ref[1, :, 0:1]
        di = di_ref[0, :, 0:1] + di_ref[1, :, 0:1]
        no = lax.rsqrt(jnp.maximum(do, 1.0))
        ni = lax.rsqrt(jnp.maximum(di, 1.0))
        no_ref[...] = no
        ni_ref[...] = ni
        hs_ref[...] = x_ref[...] * no

    return pl.pallas_call(
        body,
        grid=(G,),
        in_specs=[
            pl.BlockSpec((BN, D), lambda i: (i, 0)),
            pl.BlockSpec((2, BN, D), lambda i: (0, i, 0)),
            pl.BlockSpec((2, BN, D), lambda i: (0, i, 0)),
        ],
        out_specs=[
            pl.BlockSpec((BN, 1), lambda i: (i, 0)),
            pl.BlockSpec((BN, 1), lambda i: (i, 0)),
            pl.BlockSpec((BN, D), lambda i: (i, 0)),
        ],
        out_shape=[
            jax.ShapeDtypeStruct((N, 1), jnp.float32),
            jax.ShapeDtypeStruct((N, 1), jnp.float32),
            jax.ShapeDtypeStruct((N, D), jnp.float32),
        ],
    )(x, dop, dip)


def _layer(aggp, ni, no, W, b, N, D):
    BN = 400
    G = N // BN

    def body(a_ref, ni_ref, no_ref, w_ref, b_ref, o_ref):
        a = (a_ref[0] + a_ref[1]) * ni_ref[...]
        h = jnp.dot(a, w_ref[...], preferred_element_type=jnp.float32)
        h = h + b_ref[...]
        o_ref[...] = jnp.maximum(h, 0.0) * no_ref[...]

    return pl.pallas_call(
        body,
        grid=(G,),
        in_specs=[
            pl.BlockSpec((2, BN, D), lambda i: (0, i, 0)),
            pl.BlockSpec((BN, 1), lambda i: (i, 0)),
            pl.BlockSpec((BN, 1), lambda i: (i, 0)),
            pl.BlockSpec((D, D), lambda i: (0, 0)),
            pl.BlockSpec((1, D), lambda i: (0, 0)),
        ],
        out_specs=pl.BlockSpec((BN, D), lambda i: (i, 0)),
        out_shape=jax.ShapeDtypeStruct((N, D), jnp.float32),
    )(aggp, ni, no, W, b.reshape(1, D))


def _final(aggp, ni, W, b, N, D):
    BN = 400
    G = N // BN

    def body(a_ref, ni_ref, w_ref, b_ref, o_ref, acc_ref):
        i = pl.program_id(0)
        a = (a_ref[0] + a_ref[1]) * ni_ref[...]
        p = jnp.sum(a, axis=0, keepdims=True)

        @pl.when(i == 0)
        def _():
            acc_ref[...] = p

        @pl.when(i > 0)
        def _():
            acc_ref[...] = acc_ref[...] + p

        @pl.when(i == G - 1)
        def _():
            v = acc_ref[...] * (1.0 / N)
            o_ref[...] = jnp.dot(v, w_ref[...],
                                 preferred_element_type=jnp.float32) + b_ref[...]

    return pl.pallas_call(
        body,
        grid=(G,),
        in_specs=[
            pl.BlockSpec((2, BN, D), lambda i: (0, i, 0)),
            pl.BlockSpec((BN, 1), lambda i: (i, 0)),
            pl.BlockSpec((D, D), lambda i: (0, 0)),
            pl.BlockSpec((1, D), lambda i: (0, 0)),
        ],
        out_specs=pl.BlockSpec((1, D), lambda i: (0, 0)),
        out_shape=jax.ShapeDtypeStruct((1, D), jnp.float32),
        scratch_shapes=[pltpu.VMEM((1, D), jnp.float32)],
    )(aggp, ni, W, b.reshape(1, D))


def kernel(x, edge_index, W1, b1, W2, b2, W3, b3, W4, b4):
    N, D = x.shape
    E = edge_index.shape[1]
    src = edge_index[0].astype(jnp.int32)
    dst = edge_index[1].astype(jnp.int32)

    zero_nd = jnp.zeros((N, D), jnp.float32)
    ones_ch = jnp.ones((_CH, D), jnp.float32)

    cnt_fn = _make_cnt(N, E, D)
    dop = cnt_fn(src, ones_ch, zero_nd).reshape(_NC, N, D)
    dip = cnt_fn(dst, ones_ch, zero_nd).reshape(_NC, N, D)
    no, ni, hs = _prep(x, dop, dip, N, D)

    agg_fn = _make_agg(N, E, D)
    for W, b in ((W1, b1), (W2, b2), (W3, b3)):
        aggp = agg_fn(hs, src, dst, zero_nd).reshape(_NC, N, D)
        hs = _layer(aggp, ni, no, W, b, N, D)
    aggp = agg_fn(hs, src, dst, zero_nd).reshape(_NC, N, D)
    return _final(aggp, ni, W4, b4, N, D)


# speedup vs baseline: 10.3790x; 1.0503x over previous
import functools

import jax
import jax.numpy as jnp
from jax import lax
from jax.experimental import pallas as pl
from jax.experimental.pallas import tpu as pltpu
from jax.experimental.pallas import tpu_sc as plsc

_NC = 2
_NS = 16
_CH = 128


def _sc_mesh():
    return plsc.VectorSubcoreMesh(core_axis_name="c", subcore_axis_name="s")


def _make_agg(N, E, D):
    CHA = 80
    e_core = E // _NC
    e_tile = e_core // _NS
    nfull = e_tile // CHA
    tail = e_tile - nfull * CHA
    rows_tile = (N // _NS) // 8 * 8
    rem = N - _NS * rows_tile
    epi = (nfull - 6) % 4 + 3
    nloop = (nfull - 3 - epi) // 4
    assert nfull >= 3 + epi and tail % 8 == 0
    tsz = max(tail, 8)

    @functools.partial(
        pl.kernel,
        mesh=_sc_mesh(),
        out_type=jax.ShapeDtypeStruct((_NC * N, D), jnp.float32),
        scratch_types=(
            [pltpu.VMEM((CHA,), jnp.int32)] * 4 +
            [pltpu.VMEM((CHA,), jnp.int32)] * 4 +
            [pltpu.VMEM((CHA, D), jnp.float32)] * 4 +
            [pltpu.VMEM((tsz,), jnp.int32)] * 2 +
            [pltpu.VMEM((tsz, D), jnp.float32)] +
            [pltpu.VMEM_SHARED((N, D), jnp.float32)] +
            [pltpu.SemaphoreType.DMA] * 12
        ),
    )
    def agg(h_hbm, src_hbm, dst_hbm, zero_hbm, out_hbm,
            sA, sB, sC, sD, dA, dB, dC, dD, rA, rB, rC, rD,
            st, dt, rt, acc_sh,
            siA, siB, siC, siD, sgA, sgB, sgC, sgD, ssA, ssB, ssC, ssD):
        c = lax.axis_index("c")
        s = lax.axis_index("s")
        r0 = s * rows_tile
        pltpu.sync_copy(zero_hbm.at[pl.ds(r0, rows_tile)],
                        acc_sh.at[pl.ds(r0, rows_tile)])
        if rem:
            @pl.when(s == _NS - 1)
            def _zero_rem():
                pltpu.sync_copy(zero_hbm.at[pl.ds(N - rem, rem)],
                                acc_sh.at[pl.ds(N - rem, rem)])
        plsc.subcore_barrier()
        base = c * e_core + s * e_tile

        sets = [(sA, dA, rA, siA, sgA, ssA), (sB, dB, rB, siB, sgB, ssB),
                (sC, dC, rC, siC, sgC, ssC), (sD, dD, rD, siD, sgD, ssD)]

        def fire_idx(off, t):
            pltpu.async_copy(src_hbm.at[pl.ds(off, CHA)], t[0], t[3])
            pltpu.async_copy(dst_hbm.at[pl.ds(off, CHA)], t[1], t[3])

        def wait_idx(t):
            pltpu.make_async_copy(src_hbm.at[pl.ds(0, CHA)], t[0], t[3]).wait()
            pltpu.make_async_copy(src_hbm.at[pl.ds(0, CHA)], t[1], t[3]).wait()

        def fire_gather(t):
            pltpu.async_copy(h_hbm.at[t[0]], t[2], t[4])

        def wait_gather(t):
            pltpu.make_async_copy(h_hbm.at[pl.ds(0, CHA)], t[2], t[4]).wait()

        def fire_scatter(t):
            pltpu.async_copy(t[2], acc_sh.at[t[1]], t[5], add=True)

        def wait_scatter(t):
            pltpu.make_async_copy(h_hbm.at[pl.ds(0, CHA)], t[2], t[5]).wait()

        fire_idx(base, sets[0])
        fire_idx(base + CHA, sets[1])
        wait_idx(sets[0])
        fire_gather(sets[0])
        fire_idx(base + 2 * CHA, sets[2])
        wait_idx(sets[1])
        fire_gather(sets[1])
        wait_gather(sets[0])
        fire_scatter(sets[0])
        fire_idx(base + 3 * CHA, sets[3])
        wait_idx(sets[2])
        fire_gather(sets[2])
        wait_gather(sets[1])
        fire_scatter(sets[1])
        wait_scatter(sets[0])
        fire_idx(base + 4 * CHA, sets[0])

        def body(k, carry):
            c0 = 3 + 4 * k
            for m in range(4):
                ch = c0 + m
                t = sets[(3 + m) % 4]
                tg = sets[(2 + m) % 4]
                tw = sets[(1 + m) % 4]
                wait_idx(t)
                fire_gather(t)
                wait_gather(tg)
                fire_scatter(tg)
                wait_scatter(tw)
                fire_idx(base + (ch + 2) * CHA, tw)
            return carry

        lax.fori_loop(0, nloop, body, 0)

        cL = nfull - 1
        for ch in range(3 + 4 * nloop, nfull):
            t = sets[ch % 4]
            tg = sets[(ch - 1) % 4]
            tw = sets[(ch - 2) % 4]
            wait_idx(t)
            fire_gather(t)
            wait_gather(tg)
            fire_scatter(tg)
            wait_scatter(tw)
            if ch + 2 <= cL:
                fire_idx(base + (ch + 2) * CHA, tw)
        wait_gather(sets[cL % 4])
        fire_scatter(sets[cL % 4])
        wait_scatter(sets[(cL - 1) % 4])
        wait_scatter(sets[cL % 4])

        if tail:
            toff = base + nfull * CHA
            pltpu.sync_copy(src_hbm.at[pl.ds(toff, tail)], st)
            pltpu.sync_copy(dst_hbm.at[pl.ds(toff, tail)], dt)
            pltpu.async_copy(h_hbm.at[st], rt, sgA)
            pltpu.make_async_copy(h_hbm.at[pl.ds(0, tail)], rt, sgA).wait()
            pltpu.sync_copy(rt, acc_sh.at[dt], add=True)

        plsc.subcore_barrier()
        pltpu.sync_copy(acc_sh.at[pl.ds(r0, rows_tile)],
                        out_hbm.at[pl.ds(c * N + r0, rows_tile)])
        if rem:
            @pl.when(s == _NS - 1)
            def _out_rem():
                pltpu.sync_copy(acc_sh.at[pl.ds(N - rem, rem)],
                                out_hbm.at[pl.ds(c * N + N - rem, rem)])

    return agg


def _make_cnt(N, E, D):
    e_core = E // _NC
    e_tile = e_core // _NS
    nfull = e_tile // _CH
    tail = e_tile - nfull * _CH
    rows_tile = (N // _NS) // 8 * 8
    rem = N - _NS * rows_tile
    assert nfull >= 3 and nfull % 2 == 0 and tail % 8 == 0

    @functools.partial(
        pl.kernel,
        mesh=_sc_mesh(),
        out_type=jax.ShapeDtypeStruct((_NC * N, D), jnp.float32),
        scratch_types=[
            pltpu.VMEM((_CH,), jnp.int32),
            pltpu.VMEM((_CH,), jnp.int32),
            pltpu.VMEM((tail,), jnp.int32),
            pltpu.VMEM((_CH, D), jnp.float32),
            pltpu.VMEM((tail, D), jnp.float32),
            pltpu.VMEM_SHARED((N, D), jnp.float32),
            pltpu.SemaphoreType.DMA,
            pltpu.SemaphoreType.DMA,
        ],
    )
    def cnt(idx_hbm, ones_hbm, zero_hbm, out_hbm,
            d0, d1, dt, rows_v, rows_t, acc_sh, si0, si1):
        c = lax.axis_index("c")
        s = lax.axis_index("s")
        r0 = s * rows_tile
        pltpu.sync_copy(zero_hbm.at[pl.ds(r0, rows_tile)],
                        acc_sh.at[pl.ds(r0, rows_tile)])
        if rem:
            @pl.when(s == _NS - 1)
            def _zero_rem():
                pltpu.sync_copy(zero_hbm.at[pl.ds(N - rem, rem)],
                                acc_sh.at[pl.ds(N - rem, rem)])
        pltpu.sync_copy(ones_hbm, rows_v)
        if tail:
            pltpu.sync_copy(ones_hbm.at[pl.ds(0, tail)], rows_t)
        plsc.subcore_barrier()
        base = c * e_core + s * e_tile

        def wait_idx(d_b, sem):
            pltpu.make_async_copy(idx_hbm.at[pl.ds(0, _CH)], d_b, sem).wait()

        def half(next_off, d_c, si_c, d_o, si_o, prefetch=True):
            wait_idx(d_c, si_c)
            if prefetch:
                pltpu.async_copy(idx_hbm.at[pl.ds(next_off, _CH)], d_o, si_o)
            pltpu.sync_copy(rows_v, acc_sh.at[d_c], add=True)

        pltpu.async_copy(idx_hbm.at[pl.ds(base, _CH)], d0, si0)
        half(base + _CH, d0, si0, d1, si1)

        def body(j, carry):
            off1 = base + (2 * j + 1) * _CH
            half(off1 + _CH, d1, si1, d0, si0)
            half(off1 + 2 * _CH, d0, si0, d1, si1)
            return carry

        lax.fori_loop(0, (nfull - 2) // 2, body, 0)

        half(0, d1, si1, d0, si0, prefetch=False)
        if tail:
            toff = base + nfull * _CH
            pltpu.sync_copy(idx_hbm.at[pl.ds(toff, tail)], dt)
            pltpu.sync_copy(rows_t, acc_sh.at[dt], add=True)
        plsc.subcore_barrier()
        pltpu.sync_copy(acc_sh.at[pl.ds(r0, rows_tile)],
                        out_hbm.at[pl.ds(c * N + r0, rows_tile)])
        if rem:
            @pl.when(s == _NS - 1)
            def _out_rem():
                pltpu.sync_copy(acc_sh.at[pl.ds(N - rem, rem)],
                                out_hbm.at[pl.ds(c * N + N - rem, rem)])

    return cnt


def _prep(x, dop, dip, N, D):
    BN = 400
    G = N // BN

    def body(x_ref, do_ref, di_ref, no_ref, ni_ref, hs_ref):
        do = do_ref[0, :, 0:1] + do_ref[1, :, 0:1]
        di = di_ref[0, :, 0:1] + di_ref[1, :, 0:1]
        no = lax.rsqrt(jnp.maximum(do, 1.0))
        ni = lax.rsqrt(jnp.maximum(di, 1.0))
        no_ref[...] = no
        ni_ref[...] = ni
        hs_ref[...] = x_ref[...] * no

    return pl.pallas_call(
        body,
        grid=(G,),
        in_specs=[
            pl.BlockSpec((BN, D), lambda i: (i, 0)),
            pl.BlockSpec((2, BN, D), lambda i: (0, i, 0)),
            pl.BlockSpec((2, BN, D), lambda i: (0, i, 0)),
        ],
        out_specs=[
            pl.BlockSpec((BN, 1), lambda i: (i, 0)),
            pl.BlockSpec((BN, 1), lambda i: (i, 0)),
            pl.BlockSpec((BN, D), lambda i: (i, 0)),
        ],
        out_shape=[
            jax.ShapeDtypeStruct((N, 1), jnp.float32),
            jax.ShapeDtypeStruct((N, 1), jnp.float32),
            jax.ShapeDtypeStruct((N, D), jnp.float32),
        ],
    )(x, dop, dip)


def _layer(aggp, ni, no, W, b, N, D):
    BN = 400
    G = N // BN

    def body(a_ref, ni_ref, no_ref, w_ref, b_ref, o_ref):
        a = (a_ref[0] + a_ref[1]) * ni_ref[...]
        h = jnp.dot(a, w_ref[...], preferred_element_type=jnp.float32)
        h = h + b_ref[...]
        o_ref[...] = jnp.maximum(h, 0.0) * no_ref[...]

    return pl.pallas_call(
        body,
        grid=(G,),
        in_specs=[
            pl.BlockSpec((2, BN, D), lambda i: (0, i, 0)),
            pl.BlockSpec((BN, 1), lambda i: (i, 0)),
            pl.BlockSpec((BN, 1), lambda i: (i, 0)),
            pl.BlockSpec((D, D), lambda i: (0, 0)),
            pl.BlockSpec((1, D), lambda i: (0, 0)),
        ],
        out_specs=pl.BlockSpec((BN, D), lambda i: (i, 0)),
        out_shape=jax.ShapeDtypeStruct((N, D), jnp.float32),
    )(aggp, ni, no, W, b.reshape(1, D))


def _final(aggp, ni, W, b, N, D):
    BN = 400
    G = N // BN

    def body(a_ref, ni_ref, w_ref, b_ref, o_ref, acc_ref):
        i = pl.program_id(0)
        a = (a_ref[0] + a_ref[1]) * ni_ref[...]
        p = jnp.sum(a, axis=0, keepdims=True)

        @pl.when(i == 0)
        def _():
            acc_ref[...] = p

        @pl.when(i > 0)
        def _():
            acc_ref[...] = acc_ref[...] + p

        @pl.when(i == G - 1)
        def _():
            v = acc_ref[...] * (1.0 / N)
            o_ref[...] = jnp.dot(v, w_ref[...],
                                 preferred_element_type=jnp.float32) + b_ref[...]

    return pl.pallas_call(
        body,
        grid=(G,),
        in_specs=[
            pl.BlockSpec((2, BN, D), lambda i: (0, i, 0)),
            pl.BlockSpec((BN, 1), lambda i: (i, 0)),
            pl.BlockSpec((D, D), lambda i: (0, 0)),
            pl.BlockSpec((1, D), lambda i: (0, 0)),
        ],
        out_specs=pl.BlockSpec((1, D), lambda i: (0, 0)),
        out_shape=jax.ShapeDtypeStruct((1, D), jnp.float32),
        scratch_shapes=[pltpu.VMEM((1, D), jnp.float32)],
    )(aggp, ni, W, b.reshape(1, D))


def kernel(x, edge_index, W1, b1, W2, b2, W3, b3, W4, b4):
    N, D = x.shape
    E = edge_index.shape[1]
    src = edge_index[0].astype(jnp.int32)
    dst = edge_index[1].astype(jnp.int32)

    zero_nd = jnp.zeros((N, D), jnp.float32)
    ones_ch = jnp.ones((_CH, D), jnp.float32)

    cnt_fn = _make_cnt(N, E, D)
    dop = cnt_fn(src, ones_ch, zero_nd).reshape(_NC, N, D)
    dip = cnt_fn(dst, ones_ch, zero_nd).reshape(_NC, N, D)
    no, ni, hs = _prep(x, dop, dip, N, D)

    agg_fn = _make_agg(N, E, D)
    for W, b in ((W1, b1), (W2, b2), (W3, b3)):
        aggp = agg_fn(hs, src, dst, zero_nd).reshape(_NC, N, D)
        hs = _layer(aggp, ni, no, W, b, N, D)
    aggp = agg_fn(hs, src, dst, zero_nd).reshape(_NC, N, D)
    return _final(aggp, ni, W4, b4, N, D)


# speedup vs baseline: 11.1856x; 1.0777x over previous
import functools

import jax
import jax.numpy as jnp
from jax import lax
from jax.experimental import pallas as pl
from jax.experimental.pallas import tpu as pltpu
from jax.experimental.pallas import tpu_sc as plsc

_NC = 2
_NS = 16
_CH = 128


def _sc_mesh():
    return plsc.VectorSubcoreMesh(core_axis_name="c", subcore_axis_name="s")


def _make_agg(N, E, D):
    CHA = 80
    e_core = E // _NC
    e_tile = e_core // _NS
    nfull = e_tile // CHA
    tail = e_tile - nfull * CHA
    rows_tile = (N // _NS) // 8 * 8
    rem = N - _NS * rows_tile
    epi = (nfull - 6) % 4 + 3
    nloop = (nfull - 3 - epi) // 4
    assert nfull >= 3 + epi and tail % 8 == 0
    tsz = max(tail, 8)

    @functools.partial(
        pl.kernel,
        mesh=_sc_mesh(),
        out_type=jax.ShapeDtypeStruct((_NC * N, D), jnp.float32),
        scratch_types=(
            [pltpu.VMEM((CHA,), jnp.int32)] * 4 +
            [pltpu.VMEM((CHA,), jnp.int32)] * 4 +
            [pltpu.VMEM((CHA, D), jnp.float32)] * 4 +
            [pltpu.VMEM((tsz,), jnp.int32)] * 2 +
            [pltpu.VMEM((tsz, D), jnp.float32)] +
            [pltpu.VMEM_SHARED((N, D), jnp.float32)] +
            [pltpu.SemaphoreType.DMA] * 12
        ),
    )
    def agg(h_hbm, src_hbm, dst_hbm, zero_hbm, out_hbm,
            sA, sB, sC, sD, dA, dB, dC, dD, rA, rB, rC, rD,
            st, dt, rt, acc_sh,
            siA, siB, siC, siD, sgA, sgB, sgC, sgD, ssA, ssB, ssC, ssD):
        c = lax.axis_index("c")
        s = lax.axis_index("s")
        r0 = s * rows_tile
        pltpu.sync_copy(zero_hbm.at[pl.ds(r0, rows_tile)],
                        acc_sh.at[pl.ds(r0, rows_tile)])
        if rem:
            @pl.when(s == _NS - 1)
            def _zero_rem():
                pltpu.sync_copy(zero_hbm.at[pl.ds(N - rem, rem)],
                                acc_sh.at[pl.ds(N - rem, rem)])
        plsc.subcore_barrier()
        base = c * e_core + s * e_tile

        sets = [(sA, dA, rA, siA, sgA, ssA), (sB, dB, rB, siB, sgB, ssB),
                (sC, dC, rC, siC, sgC, ssC), (sD, dD, rD, siD, sgD, ssD)]

        def fire_idx(off, t):
            pltpu.async_copy(src_hbm.at[pl.ds(off, CHA)], t[0], t[3])
            pltpu.async_copy(dst_hbm.at[pl.ds(off, CHA)], t[1], t[3])

        def wait_idx(t):
            pltpu.make_async_copy(src_hbm.at[pl.ds(0, CHA)], t[0], t[3]).wait()
            pltpu.make_async_copy(src_hbm.at[pl.ds(0, CHA)], t[1], t[3]).wait()

        def fire_gather(t):
            pltpu.async_copy(h_hbm.at[t[0]], t[2], t[4])

        def wait_gather(t):
            pltpu.make_async_copy(h_hbm.at[pl.ds(0, CHA)], t[2], t[4]).wait()

        def fire_scatter(t):
            pltpu.async_copy(t[2], acc_sh.at[t[1]], t[5], add=True)

        def wait_scatter(t):
            pltpu.make_async_copy(h_hbm.at[pl.ds(0, CHA)], t[2], t[5]).wait()

        fire_idx(base, sets[0])
        fire_idx(base + CHA, sets[1])
        wait_idx(sets[0])
        fire_gather(sets[0])
        fire_idx(base + 2 * CHA, sets[2])
        wait_idx(sets[1])
        fire_gather(sets[1])
        wait_gather(sets[0])
        fire_scatter(sets[0])
        fire_idx(base + 3 * CHA, sets[3])
        wait_idx(sets[2])
        fire_gather(sets[2])
        wait_gather(sets[1])
        fire_scatter(sets[1])
        wait_scatter(sets[0])
        fire_idx(base + 4 * CHA, sets[0])

        def body(k, carry):
            c0 = 3 + 4 * k
            for m in range(4):
                ch = c0 + m
                t = sets[(3 + m) % 4]
                tg = sets[(2 + m) % 4]
                tw = sets[(1 + m) % 4]
                wait_idx(t)
                fire_gather(t)
                wait_gather(tg)
                fire_scatter(tg)
                wait_scatter(tw)
                fire_idx(base + (ch + 2) * CHA, tw)
            return carry

        lax.fori_loop(0, nloop, body, 0)

        cL = nfull - 1
        for ch in range(3 + 4 * nloop, nfull):
            t = sets[ch % 4]
            tg = sets[(ch - 1) % 4]
            tw = sets[(ch - 2) % 4]
            wait_idx(t)
            fire_gather(t)
            wait_gather(tg)
            fire_scatter(tg)
            wait_scatter(tw)
            if ch + 2 <= cL:
                fire_idx(base + (ch + 2) * CHA, tw)
        wait_gather(sets[cL % 4])
        fire_scatter(sets[cL % 4])
        wait_scatter(sets[(cL - 1) % 4])
        wait_scatter(sets[cL % 4])

        if tail:
            toff = base + nfull * CHA
            pltpu.sync_copy(src_hbm.at[pl.ds(toff, tail)], st)
            pltpu.sync_copy(dst_hbm.at[pl.ds(toff, tail)], dt)
            pltpu.async_copy(h_hbm.at[st], rt, sgA)
            pltpu.make_async_copy(h_hbm.at[pl.ds(0, tail)], rt, sgA).wait()
            pltpu.sync_copy(rt, acc_sh.at[dt], add=True)

        plsc.subcore_barrier()
        pltpu.sync_copy(acc_sh.at[pl.ds(r0, rows_tile)],
                        out_hbm.at[pl.ds(c * N + r0, rows_tile)])
        if rem:
            @pl.when(s == _NS - 1)
            def _out_rem():
                pltpu.sync_copy(acc_sh.at[pl.ds(N - rem, rem)],
                                out_hbm.at[pl.ds(c * N + N - rem, rem)])

    return agg


def _make_cnt(N, E, D):
    e_core = E // _NC
    e_tile = e_core // _NS
    nfull = e_tile // _CH
    tail = e_tile - nfull * _CH
    rows_tile = (N // _NS) // 8 * 8
    rem = N - _NS * rows_tile
    assert nfull >= 3 and nfull % 2 == 0 and tail % 8 == 0

    @functools.partial(
        pl.kernel,
        mesh=_sc_mesh(),
        out_type=jax.ShapeDtypeStruct((_NC * N, D), jnp.float32),
        scratch_types=[
            pltpu.VMEM((_CH,), jnp.int32),
            pltpu.VMEM((_CH,), jnp.int32),
            pltpu.VMEM((tail,), jnp.int32),
            pltpu.VMEM((_CH, D), jnp.float32),
            pltpu.VMEM((tail, D), jnp.float32),
            pltpu.VMEM_SHARED((N, D), jnp.float32),
            pltpu.SemaphoreType.DMA,
            pltpu.SemaphoreType.DMA,
        ],
    )
    def cnt(idx_hbm, ones_hbm, zero_hbm, out_hbm,
            d0, d1, dt, rows_v, rows_t, acc_sh, si0, si1):
        c = lax.axis_index("c")
        s = lax.axis_index("s")
        r0 = s * rows_tile
        pltpu.sync_copy(zero_hbm.at[pl.ds(r0, rows_tile)],
                        acc_sh.at[pl.ds(r0, rows_tile)])
        if rem:
            @pl.when(s == _NS - 1)
            def _zero_rem():
                pltpu.sync_copy(zero_hbm.at[pl.ds(N - rem, rem)],
                                acc_sh.at[pl.ds(N - rem, rem)])
        pltpu.sync_copy(ones_hbm, rows_v)
        if tail:
            pltpu.sync_copy(ones_hbm.at[pl.ds(0, tail)], rows_t)
        plsc.subcore_barrier()
        base = c * e_core + s * e_tile

        def wait_idx(d_b, sem):
            pltpu.make_async_copy(idx_hbm.at[pl.ds(0, _CH)], d_b, sem).wait()

        def half(next_off, d_c, si_c, d_o, si_o, prefetch=True):
            wait_idx(d_c, si_c)
            if prefetch:
                pltpu.async_copy(idx_hbm.at[pl.ds(next_off, _CH)], d_o, si_o)
            pltpu.sync_copy(rows_v, acc_sh.at[d_c], add=True)

        pltpu.async_copy(idx_hbm.at[pl.ds(base, _CH)], d0, si0)
        half(base + _CH, d0, si0, d1, si1)

        def body(j, carry):
            off1 = base + (2 * j + 1) * _CH
            half(off1 + _CH, d1, si1, d0, si0)
            half(off1 + 2 * _CH, d0, si0, d1, si1)
            return carry

        lax.fori_loop(0, (nfull - 2) // 2, body, 0)

        half(0, d1, si1, d0, si0, prefetch=False)
        if tail:
            toff = base + nfull * _CH
            pltpu.sync_copy(idx_hbm.at[pl.ds(toff, tail)], dt)
            pltpu.sync_copy(rows_t, acc_sh.at[dt], add=True)
        plsc.subcore_barrier()
        pltpu.sync_copy(acc_sh.at[pl.ds(r0, rows_tile)],
                        out_hbm.at[pl.ds(c * N + r0, rows_tile)])
        if rem:
            @pl.when(s == _NS - 1)
            def _out_rem():
                pltpu.sync_copy(acc_sh.at[pl.ds(N - rem, rem)],
                                out_hbm.at[pl.ds(c * N + N - rem, rem)])

    return cnt


def _prep(x, dop, dip, N, D):
    BN = 2000
    G = N // BN

    def body(x_ref, do_ref, di_ref, no_ref, ni_ref, hs_ref):
        do = do_ref[0, :, 0:1] + do_ref[1, :, 0:1]
        di = di_ref[0, :, 0:1] + di_ref[1, :, 0:1]
        no = lax.rsqrt(jnp.maximum(do, 1.0))
        ni = lax.rsqrt(jnp.maximum(di, 1.0))
        no_ref[...] = no
        ni_ref[...] = ni
        hs_ref[...] = x_ref[...] * no

    return pl.pallas_call(
        body,
        grid=(G,),
        in_specs=[
            pl.BlockSpec((BN, D), lambda i: (i, 0)),
            pl.BlockSpec((2, BN, D), lambda i: (0, i, 0)),
            pl.BlockSpec((2, BN, D), lambda i: (0, i, 0)),
        ],
        out_specs=[
            pl.BlockSpec((BN, 1), lambda i: (i, 0)),
            pl.BlockSpec((BN, 1), lambda i: (i, 0)),
            pl.BlockSpec((BN, D), lambda i: (i, 0)),
        ],
        out_shape=[
            jax.ShapeDtypeStruct((N, 1), jnp.float32),
            jax.ShapeDtypeStruct((N, 1), jnp.float32),
            jax.ShapeDtypeStruct((N, D), jnp.float32),
        ],
    )(x, dop, dip)


def _layer(aggp, ni, no, W, b, N, D):
    BN = 2000
    G = N // BN

    def body(a_ref, ni_ref, no_ref, w_ref, b_ref, o_ref):
        a = (a_ref[0] + a_ref[1]) * ni_ref[...]
        h = jnp.dot(a, w_ref[...], preferred_element_type=jnp.float32)
        h = h + b_ref[...]
        o_ref[...] = jnp.maximum(h, 0.0) * no_ref[...]

    return pl.pallas_call(
        body,
        grid=(G,),
        in_specs=[
            pl.BlockSpec((2, BN, D), lambda i: (0, i, 0)),
            pl.BlockSpec((BN, 1), lambda i: (i, 0)),
            pl.BlockSpec((BN, 1), lambda i: (i, 0)),
            pl.BlockSpec((D, D), lambda i: (0, 0)),
            pl.BlockSpec((1, D), lambda i: (0, 0)),
        ],
        out_specs=pl.BlockSpec((BN, D), lambda i: (i, 0)),
        out_shape=jax.ShapeDtypeStruct((N, D), jnp.float32),
    )(aggp, ni, no, W, b.reshape(1, D))


def _final(aggp, ni, W, b, N, D):
    BN = 2000
    G = N // BN

    def body(a_ref, ni_ref, w_ref, b_ref, o_ref, acc_ref):
        i = pl.program_id(0)
        a = (a_ref[0] + a_ref[1]) * ni_ref[...]
        p = jnp.sum(a, axis=0, keepdims=True)

        @pl.when(i == 0)
        def _():
            acc_ref[...] = p

        @pl.when(i > 0)
        def _():
            acc_ref[...] = acc_ref[...] + p

        @pl.when(i == G - 1)
        def _():
            v = acc_ref[...] * (1.0 / N)
            o_ref[...] = jnp.dot(v, w_ref[...],
                                 preferred_element_type=jnp.float32) + b_ref[...]

    return pl.pallas_call(
        body,
        grid=(G,),
        in_specs=[
            pl.BlockSpec((2, BN, D), lambda i: (0, i, 0)),
            pl.BlockSpec((BN, 1), lambda i: (i, 0)),
            pl.BlockSpec((D, D), lambda i: (0, 0)),
            pl.BlockSpec((1, D), lambda i: (0, 0)),
        ],
        out_specs=pl.BlockSpec((1, D), lambda i: (0, 0)),
        out_shape=jax.ShapeDtypeStruct((1, D), jnp.float32),
        scratch_shapes=[pltpu.VMEM((1, D), jnp.float32)],
    )(aggp, ni, W, b.reshape(1, D))


def kernel(x, edge_index, W1, b1, W2, b2, W3, b3, W4, b4):
    N, D = x.shape
    E = edge_index.shape[1]
    src = edge_index[0].astype(jnp.int32)
    dst = edge_index[1].astype(jnp.int32)

    zero_nd = jnp.zeros((N, D), jnp.float32)
    ones_ch = jnp.ones((_CH, D), jnp.float32)

    cnt_fn = _make_cnt(N, E, D)
    dop = cnt_fn(src, ones_ch, zero_nd).reshape(_NC, N, D)
    dip = cnt_fn(dst, ones_ch, zero_nd).reshape(_NC, N, D)
    no, ni, hs = _prep(x, dop, dip, N, D)

    agg_fn = _make_agg(N, E, D)
    for W, b in ((W1, b1), (W2, b2), (W3, b3)):
        aggp = agg_fn(hs, src, dst, zero_nd).reshape(_NC, N, D)
        hs = _layer(aggp, ni, no, W, b, N, D)
    aggp = agg_fn(hs, src, dst, zero_nd).reshape(_NC, N, D)
    return _final(aggp, ni, W4, b4, N, D)
